# Initial kernel scaffold; baseline (speedup 1.0000x reference)
import jax
import jax.numpy as jnp
from jax.experimental import pallas as pl


def kernel(x, segment_ids, pos, Wq, bq, Wk, bk, Wv, bv, Wo, bo):
    raise NotImplementedError("write your pallas kernel here")


# speedup vs baseline: 7.9742x; 7.9742x over previous
import jax
import jax.numpy as jnp
from jax.experimental import pallas as pl
from jax.experimental.pallas import tpu as pltpu

SEQ = 2048
EMBED = 1024
HEADS = 16
HEAD_DIM = EMBED // HEADS
TILE = 256
NUM_TILES = SEQ // TILE

_DN = (((1,), (1,)), ((), ()))


def _dotT(a, b):
    return jax.lax.dot_general(a, b, _DN, preferred_element_type=jnp.float32)


def _body(x_ref, seg_ref, pos_ref, wq_ref, bq_ref, wk_ref, bk_ref,
          wv_ref, bv_ref, wo_ref, bo_ref, out_ref,
          k_scr, v_scr, num_scr, den_scr, acc_ref, vseg_ref):
    pos = pos_ref[0]
    seg = seg_ref[...]
    flat_idx = (jax.lax.broadcasted_iota(jnp.int32, seg.shape, 0) * 128
                + jax.lax.broadcasted_iota(jnp.int32, seg.shape, 1))
    sid = jnp.sum(jnp.where(flat_idx == pos, seg, 0))
    start = jnp.sum((seg < sid).astype(jnp.int32))
    end = jnp.sum((seg <= sid).astype(jnp.int32))
    length = end - start
    t0 = start // TILE
    t1 = (end - 1) // TILE + 1

    sum_x = jnp.sum(x_ref[...], axis=0, keepdims=True)
    vtot = _dotT(sum_x, wv_ref[...]) + SEQ * bv_ref[...]

    acc_ref[...] = jnp.zeros_like(acc_ref)
    vseg_ref[...] = jnp.zeros_like(vseg_ref)

    def kv_body(t, _):
        rows = x_ref[pl.ds(t * TILE, TILE), :]
        kt = _dotT(rows, wk_ref[...]) + bk_ref[...]
        vt = _dotT(rows, wv_ref[...]) + bv_ref[...]
        k_scr[pl.ds(t * TILE, TILE), :] = kt
        v_scr[pl.ds(t * TILE, TILE), :] = vt
        gidx = t * TILE + jax.lax.broadcasted_iota(jnp.int32, (TILE, 1), 0)
        rmask = (gidx >= start) & (gidx < end)
        vseg_ref[...] += jnp.sum(jnp.where(rmask, vt, 0.0), axis=0,
                                 keepdims=True)
        return 0

    jax.lax.fori_loop(t0, t1, kv_body, 0)

    comp_v = vtot - vseg_ref[...]
    comp_d = (SEQ - length).astype(jnp.float32)

    def ti_body(ti, _):
        rows = x_ref[pl.ds(ti * TILE, TILE), :]
        qt = _dotT(rows, wq_ref[...]) + bq_ref[...]
        num_scr[...] = jnp.zeros_like(num_scr)
        den_scr[...] = jnp.zeros_like(den_scr)

        def tj_body(tj, _):
            kt = k_scr[pl.ds(tj * TILE, TILE), :]
            vt = v_scr[pl.ds(tj * TILE, TILE), :]
            cidx = tj * TILE + jax.lax.broadcasted_iota(
                jnp.int32, (TILE, TILE), 1)
            cmask = (cidx >= start) & (cidx < end)
            for h in range(HEADS):
                sl = slice(h * HEAD_DIM, (h + 1) * HEAD_DIM)
                s = _dotT(qt[:, sl], kt[:, sl])
                e = jnp.where(cmask, jnp.exp(s), 0.0)
                den_scr[:, h:h + 1] += jnp.sum(e, axis=1, keepdims=True)
                num_scr[:, sl] += jax.lax.dot_general(
                    e, vt[:, sl], (((1,), (0,)), ((), ())),
                    preferred_element_type=jnp.float32)
            return 0

        jax.lax.fori_loop(t0, t1, tj_body, 0)

        gidx = ti * TILE + jax.lax.broadcasted_iota(jnp.int32, (TILE, 1), 0)
        rmask = (gidx >= start) & (gidx < end)
        for h in range(HEADS):
            sl = slice(h * HEAD_DIM, (h + 1) * HEAD_DIM)
            w = ((num_scr[:, sl] + comp_v[:, sl])
                 / (den_scr[:, h:h + 1] + comp_d))
            acc_ref[:, sl] += jnp.sum(jnp.where(rmask, w, 0.0), axis=0,
                                      keepdims=True)
        return 0

    jax.lax.fori_loop(t0, t1, ti_body, 0)

    mean_w = acc_ref[...] / length.astype(jnp.float32)
    out_ref[...] = _dotT(mean_w, wo_ref[...]) + bo_ref[...]


def kernel(x, segment_ids, pos, Wq, bq, Wk, bk, Wv, bv, Wo, bo):
    seg2d = segment_ids.astype(jnp.int32).reshape(16, 128)
    pos_arr = jnp.asarray(pos, jnp.int32).reshape(1)
    out = pl.pallas_call(
        _body,
        out_shape=jax.ShapeDtypeStruct((1, EMBED), jnp.float32),
        in_specs=[
            pl.BlockSpec(memory_space=pltpu.VMEM),
            pl.BlockSpec(memory_space=pltpu.VMEM),
            pl.BlockSpec(memory_space=pltpu.SMEM),
            pl.BlockSpec(memory_space=pltpu.VMEM),
            pl.BlockSpec(memory_space=pltpu.VMEM),
            pl.BlockSpec(memory_space=pltpu.VMEM),
            pl.BlockSpec(memory_space=pltpu.VMEM),
            pl.BlockSpec(memory_space=pltpu.VMEM),
            pl.BlockSpec(memory_space=pltpu.VMEM),
            pl.BlockSpec(memory_space=pltpu.VMEM),
            pl.BlockSpec(memory_space=pltpu.VMEM),
        ],
        out_specs=pl.BlockSpec(memory_space=pltpu.VMEM),
        scratch_shapes=[
            pltpu.VMEM((SEQ, EMBED), jnp.float32),
            pltpu.VMEM((SEQ, EMBED), jnp.float32),
            pltpu.VMEM((TILE, EMBED), jnp.float32),
            pltpu.VMEM((TILE, 128), jnp.float32),
            pltpu.VMEM((1, EMBED), jnp.float32),
            pltpu.VMEM((1, EMBED), jnp.float32),
        ],
    )(x, seg2d, pos_arr,
      Wq, bq.reshape(1, EMBED), Wk, bk.reshape(1, EMBED),
      Wv, bv.reshape(1, EMBED), Wo, bo.reshape(1, EMBED))
    return out.reshape(EMBED)
